# Initial kernel scaffold; baseline (speedup 1.0000x reference)
import jax
import jax.numpy as jnp
from jax.experimental import pallas as pl


def kernel(x, edge_index, edge_weight, region_id, region_adjacency, region_area, coarse_region_similarity, W_enc, W_reg, weight_poi2region, weight_region2city):
    raise NotImplementedError("write your pallas kernel here")


# speedup vs baseline: 3.1245x; 3.1245x over previous
import functools

import jax
import jax.numpy as jnp
import numpy as np
from jax import lax
from jax.experimental import pallas as pl
from jax.experimental.pallas import tpu as pltpu
from jax.experimental.pallas import tpu_sc as plsc

_N = 10000
_E = 320000
_D = 128
_R = 500
_ER = 4000
_RP = 512

_NS = 16
_EPT = _E // _NS
_KB = 80
_NB = _EPT // _KB
_GB = _KB // 16
_WPT = 624
_XTRA = _N - _NS * _WPT
_XC = 104
_NXC = _WPT // _XC


def _sc_body(x_hbm, perm2_hbm, src_hbm, dst_hbm, w_hbm, out_hbm,
             acc_sh, src_v, psrc_v, dst_v, w_v, rows_v, xrows_v, xrows2_v,
             pidx_v, pidx2_v, sem):
    c = lax.axis_index("c")
    s = lax.axis_index("s")
    row0 = s * _WPT
    last = _NS * _WPT

    @pl.when(c == 0)
    def _():
        pltpu.sync_copy(x_hbm.at[pl.ds(row0, _WPT)], acc_sh.at[pl.ds(row0, _WPT)])

        @pl.when(s == _NS - 1)
        def _():
            pltpu.sync_copy(x_hbm.at[pl.ds(last, _XTRA)],
                            acc_sh.at[pl.ds(last, _XTRA)])

    @pl.when(c == 1)
    def _():
        pltpu.sync_copy(perm2_hbm.at[pl.ds(_N + row0, _WPT)], pidx_v)
        for i in range(_NXC):
            pltpu.async_copy(
                x_hbm.at[pidx_v.at[pl.ds(i * _XC, _XC)]], xrows_v, sem
            ).wait()
            pltpu.sync_copy(xrows_v, acc_sh.at[pl.ds(row0 + i * _XC, _XC)])

        @pl.when(s == _NS - 1)
        def _():
            pltpu.sync_copy(perm2_hbm.at[pl.ds(_N + last, _XTRA)], pidx2_v)
            pltpu.async_copy(x_hbm.at[pidx2_v], xrows2_v, sem).wait()
            pltpu.sync_copy(xrows2_v, acc_sh.at[pl.ds(last, _XTRA)])

    plsc.subcore_barrier()

    cN = c * _N

    def batch(b, carry):
        ebase = s * _EPT + b * _KB
        pltpu.sync_copy(src_hbm.at[pl.ds(ebase, _KB)], src_v)
        pltpu.sync_copy(dst_hbm.at[pl.ds(ebase, _KB)], dst_v)
        pltpu.sync_copy(w_hbm.at[pl.ds(ebase, _KB)], w_v)
        for g in range(_GB):
            sl = pl.ds(g * 16, 16)
            src_v[sl] = src_v[sl] + cN
        pltpu.async_copy(perm2_hbm.at[src_v], psrc_v, sem).wait()
        pltpu.async_copy(x_hbm.at[psrc_v], rows_v, sem).wait()

        def grp(g, carry2):
            e0 = jnp.full((16,), g * 16, dtype=jnp.int32)
            for j in range(16):
                e = g * 16 + j
                wj = plsc.load_gather(w_v, [e0 + j])
                for d2 in range(_D // 16):
                    dsl = pl.ds(d2 * 16, 16)
                    rows_v[e, dsl] = rows_v[e, dsl] * wj
            return carry2
        lax.fori_loop(0, _GB, grp, 0, unroll=False)

        pltpu.sync_copy(rows_v, acc_sh.at[dst_v], add=True)
        return carry
    lax.fori_loop(0, _NB, batch, 0, unroll=False)

    plsc.subcore_barrier()
    pltpu.sync_copy(acc_sh.at[pl.ds(row0, _WPT)],
                    out_hbm.at[c].at[pl.ds(row0, _WPT)])

    @pl.when(s == _NS - 1)
    def _():
        pltpu.sync_copy(acc_sh.at[pl.ds(last, _XTRA)],
                        out_hbm.at[c].at[pl.ds(last, _XTRA)])


def _sc_dual_segment_sum(x, perm2, src, dst, w):
    mesh = plsc.VectorSubcoreMesh(core_axis_name="c", subcore_axis_name="s")
    fn = pl.kernel(
        _sc_body,
        out_type=jax.ShapeDtypeStruct((2, _N, _D), jnp.float32),
        mesh=mesh,
        compiler_params=pltpu.CompilerParams(needs_layout_passes=False),
        scratch_types=[
            pltpu.VMEM_SHARED((_N, _D), jnp.float32),
            pltpu.VMEM((_KB,), jnp.int32),
            pltpu.VMEM((_KB,), jnp.int32),
            pltpu.VMEM((_KB,), jnp.int32),
            pltpu.VMEM((_KB,), jnp.float32),
            pltpu.VMEM((_KB, _D), jnp.float32),
            pltpu.VMEM((_XC, _D), jnp.float32),
            pltpu.VMEM((_XTRA, _D), jnp.float32),
            pltpu.VMEM((_WPT,), jnp.int32),
            pltpu.VMEM((_XTRA,), jnp.int32),
            pltpu.SemaphoreType.DMA,
        ],
    )
    return fn(x, perm2, src, dst, w)


_BN = 1000


def _k1_body(a0_ref, a1_ref, w_ref, rid_ref, pos_ref, sp_ref, sn_ref, cnt_ref):
    i = pl.program_id(0)
    W = w_ref[...]
    ep = jnp.maximum(jnp.dot(a0_ref[...], W, preferred_element_type=jnp.float32), 0.0)
    en = jnp.maximum(jnp.dot(a1_ref[...], W, preferred_element_type=jnp.float32), 0.0)
    pos_ref[...] = ep
    rid = rid_ref[0, 0, :]
    rows = lax.broadcasted_iota(jnp.int32, (_RP, _BN), 0)
    oT = (rows == rid[None, :]).astype(jnp.float32)
    cp = jnp.dot(oT, ep, preferred_element_type=jnp.float32)
    cn = jnp.dot(oT, en, preferred_element_type=jnp.float32)
    c1 = jnp.sum(oT, axis=1)

    @pl.when(i == 0)
    def _():
        sp_ref[...] = jnp.zeros_like(sp_ref)
        sn_ref[...] = jnp.zeros_like(sn_ref)
        cnt_ref[...] = jnp.zeros_like(cnt_ref)

    sp_ref[...] += cp
    sn_ref[...] += cn
    cnt_ref[...] += c1


def _tc_encode(acc0, acc1, W_enc, rid):
    return pl.pallas_call(
        _k1_body,
        grid=(_N // _BN,),
        in_specs=[
            pl.BlockSpec((_BN, _D), lambda i: (i, 0)),
            pl.BlockSpec((_BN, _D), lambda i: (i, 0)),
            pl.BlockSpec((_D, _D), lambda i: (0, 0)),
            pl.BlockSpec((1, 1, _BN), lambda i: (i, 0, 0)),
        ],
        out_specs=[
            pl.BlockSpec((_BN, _D), lambda i: (i, 0)),
            pl.BlockSpec((_RP, _D), lambda i: (0, 0)),
            pl.BlockSpec((_RP, _D), lambda i: (0, 0)),
            pl.BlockSpec((_RP,), lambda i: (0,)),
        ],
        out_shape=[
            jax.ShapeDtypeStruct((_N, _D), jnp.float32),
            jax.ShapeDtypeStruct((_RP, _D), jnp.float32),
            jax.ShapeDtypeStruct((_RP, _D), jnp.float32),
            jax.ShapeDtypeStruct((_RP,), jnp.float32),
        ],
    )(acc0, acc1, W_enc, rid)


_EC = 1000


def _k2_body(sp_ref, sn_ref, cnt_ref, rs_ref, rd_ref, w_ref, area_ref, sim_ref,
             re_ref, ne_ref, ng_ref, city_ref):
    inv = 1.0 / jnp.maximum(cnt_ref[...], 1.0)
    mp = sp_ref[...] * inv[:, None]
    mn = sn_ref[...] * inv[:, None]
    rs = rs_ref[...]
    rd = rd_ref[...]
    cols = lax.broadcasted_iota(jnp.int32, (_EC, _RP), 1)
    rows = lax.broadcasted_iota(jnp.int32, (_RP, _EC), 0)
    raggp = jnp.zeros((_RP, _D), jnp.float32)
    raggn = jnp.zeros((_RP, _D), jnp.float32)
    for ch in range(_ER // _EC):
        rs_c = rs[ch * _EC:(ch + 1) * _EC]
        rd_c = rd[ch * _EC:(ch + 1) * _EC]
        oS = (cols == rs_c[:, None]).astype(jnp.float32)
        oDT = (rows == rd_c[None, :]).astype(jnp.float32)
        gp = jnp.dot(oS, mp, preferred_element_type=jnp.float32)
        gn = jnp.dot(oS, mn, preferred_element_type=jnp.float32)
        raggp = raggp + jnp.dot(oDT, gp, preferred_element_type=jnp.float32)
        raggn = raggn + jnp.dot(oDT, gn, preferred_element_type=jnp.float32)
    W = w_ref[...]
    re = jnp.maximum(jnp.dot(mp + raggp, W, preferred_element_type=jnp.float32), 0.0)
    ne = jnp.maximum(jnp.dot(mn + raggn, W, preferred_element_type=jnp.float32), 0.0)
    re_ref[...] = re
    ne_ref[...] = ne

    area = area_ref[...]
    wgt = area / jnp.sum(area)
    city_ref[...] = jnp.tanh(jnp.sum(re * wgt[:, None], axis=0, keepdims=True))

    sim = sim_ref[...]
    ii = lax.broadcasted_iota(jnp.int32, (_RP, _RP), 1)
    jj = lax.broadcasted_iota(jnp.int32, (_RP, _RP), 0)
    mask = (sim > 0.6) & (sim < 0.8) & (ii != jj)
    masked = jnp.where(mask, sim, -jnp.inf)
    mx = jnp.max(masked, axis=1, keepdims=True)
    has = jnp.sum(mask.astype(jnp.float32), axis=1) > 0.0
    cand = jnp.min(jnp.where(masked == mx, ii, _RP), axis=1)
    fallback = lax.rem(lax.broadcasted_iota(jnp.int32, (_RP,), 0) + 1, _R)
    neg_idx = jnp.where(has, cand, fallback)
    oN = (ii == neg_idx[:, None]).astype(jnp.float32)
    ng_ref[...] = jnp.dot(oN, re, preferred_element_type=jnp.float32)


def _tc_region(sp, sn, cnt, rsrc, rdst, W_reg, area_p, sim_p):
    return pl.pallas_call(
        _k2_body,
        out_shape=[
            jax.ShapeDtypeStruct((_RP, _D), jnp.float32),
            jax.ShapeDtypeStruct((_RP, _D), jnp.float32),
            jax.ShapeDtypeStruct((_RP, _D), jnp.float32),
            jax.ShapeDtypeStruct((1, _D), jnp.float32),
        ],
    )(sp, sn, cnt, rsrc, rdst, W_reg, area_p, sim_p)


def _k3_body(rid_ref, re_ref, ng_ref, o2_ref, o3_ref):
    rid = rid_ref[0, 0, :]
    cols = lax.broadcasted_iota(jnp.int32, (_BN, _RP), 1)
    o = (cols == rid[:, None]).astype(jnp.float32)
    o2_ref[...] = jnp.dot(o, re_ref[...], preferred_element_type=jnp.float32)
    o3_ref[...] = jnp.dot(o, ng_ref[...], preferred_element_type=jnp.float32)


def _tc_gather_out(rid, re, ng):
    return pl.pallas_call(
        _k3_body,
        grid=(_N // _BN,),
        in_specs=[
            pl.BlockSpec((1, 1, _BN), lambda i: (i, 0, 0)),
            pl.BlockSpec((_RP, _D), lambda i: (0, 0)),
            pl.BlockSpec((_RP, _D), lambda i: (0, 0)),
        ],
        out_specs=[
            pl.BlockSpec((_BN, _D), lambda i: (i, 0)),
            pl.BlockSpec((_BN, _D), lambda i: (i, 0)),
        ],
        out_shape=[
            jax.ShapeDtypeStruct((_N, _D), jnp.float32),
            jax.ShapeDtypeStruct((_N, _D), jnp.float32),
        ],
    )(rid, re, ng)


def kernel(x, edge_index, edge_weight, region_id, region_adjacency, region_area,
           coarse_region_similarity, W_enc, W_reg, weight_poi2region,
           weight_region2city):
    perm = jax.random.permutation(jax.random.key(42), _N).astype(jnp.int32)
    perm2 = jnp.concatenate([jnp.arange(_N, dtype=jnp.int32), perm])
    acc = _sc_dual_segment_sum(
        x, perm2, edge_index[0], edge_index[1], edge_weight)
    rid3 = region_id.reshape(_N // _BN, 1, _BN)
    pos_poi, sp, sn, cnt = _tc_encode(acc[0], acc[1], W_enc, rid3)

    area_p = jnp.pad(region_area, (0, _RP - _R))
    sim_p = jnp.pad(coarse_region_similarity, ((0, _RP - _R), (0, _RP - _R)))
    re, ne, ng, city = _tc_region(
        sp, sn, cnt, region_adjacency[0], region_adjacency[1], W_reg,
        area_p, sim_p)
    out2, out3 = _tc_gather_out(rid3, re, ng)
    return (pos_poi, out2, out3, re[:_R], ne[:_R], city[0])


# speedup vs baseline: 7.4021x; 2.3691x over previous
import functools

import jax
import jax.numpy as jnp
import numpy as np
from jax import lax
from jax.experimental import pallas as pl
from jax.experimental.pallas import tpu as pltpu
from jax.experimental.pallas import tpu_sc as plsc

_N = 10000
_E = 320000
_D = 128
_R = 500
_ER = 4000
_RP = 512

_NS = 16
_EPT = _E // _NS
_KB = 32
_NB = _EPT // _KB
_GB = _KB // 16
_WPT = 624
_XTRA = _N - _NS * _WPT
_XC = 48
_NXC = _WPT // _XC


_RING = 5


def _sc_body(x_hbm, perm_hbm, src_hbm, dst_hbm, w_hbm, out_hbm, xp_hbm,
             acc_sh, src_v, dst_v, w_v, rows_v, xrows_v, xrows2_v,
             pidx_v, pidx2_v, *sems):
    sem_idx = sems[0:_RING]
    sem_rows = sems[_RING:2 * _RING]
    sem_scat = sems[2 * _RING:3 * _RING]
    sem = sems[3 * _RING]
    c = lax.axis_index("c")
    s = lax.axis_index("s")
    row0 = s * _WPT
    last = _NS * _WPT

    @pl.when(c == 0)
    def _():
        pltpu.sync_copy(x_hbm.at[pl.ds(row0, _WPT)], acc_sh.at[pl.ds(row0, _WPT)])

        @pl.when(s == _NS - 1)
        def _():
            pltpu.sync_copy(x_hbm.at[pl.ds(last, _XTRA)],
                            acc_sh.at[pl.ds(last, _XTRA)])

    @pl.when(c == 1)
    def _():
        pltpu.sync_copy(perm_hbm.at[pl.ds(row0, _WPT)], pidx_v)
        for i in range(_NXC):
            pltpu.async_copy(
                x_hbm.at[pidx_v.at[pl.ds(i * _XC, _XC)]], xrows_v, sem
            ).wait()
            pltpu.sync_copy(xrows_v, acc_sh.at[pl.ds(row0 + i * _XC, _XC)])
            pltpu.sync_copy(xrows_v, xp_hbm.at[pl.ds(row0 + i * _XC, _XC)])

        @pl.when(s == _NS - 1)
        def _():
            pltpu.sync_copy(perm_hbm.at[pl.ds(last, _XTRA)], pidx2_v)
            pltpu.async_copy(x_hbm.at[pidx2_v], xrows2_v, sem).wait()
            pltpu.sync_copy(xrows2_v, acc_sh.at[pl.ds(last, _XTRA)])
            pltpu.sync_copy(xrows2_v, xp_hbm.at[pl.ds(last, _XTRA)])

    plsc.subcore_barrier()

    ebase0 = s * _EPT

    def issue_idx(b, r):
        eb = ebase0 + b * _KB
        pltpu.async_copy(src_hbm.at[pl.ds(eb, _KB)], src_v.at[r], sem_idx[r])
        pltpu.async_copy(dst_hbm.at[pl.ds(eb, _KB)], dst_v.at[r], sem_idx[r])
        pltpu.async_copy(w_hbm.at[pl.ds(eb, _KB)], w_v.at[r], sem_idx[r])

    def wait_idx(b, r):
        eb = ebase0 + b * _KB
        pltpu.make_async_copy(src_hbm.at[pl.ds(eb, _KB)], src_v.at[r],
                              sem_idx[r]).wait()
        pltpu.make_async_copy(dst_hbm.at[pl.ds(eb, _KB)], dst_v.at[r],
                              sem_idx[r]).wait()
        pltpu.make_async_copy(w_hbm.at[pl.ds(eb, _KB)], w_v.at[r],
                              sem_idx[r]).wait()

    def issue_gather(r):
        @pl.when(c == 0)
        def _():
            pltpu.async_copy(x_hbm.at[src_v.at[r]], rows_v.at[r], sem_rows[r])

        @pl.when(c == 1)
        def _():
            pltpu.async_copy(xp_hbm.at[src_v.at[r]], rows_v.at[r], sem_rows[r])

    def wait_gather(r):
        @pl.when(c == 0)
        def _():
            pltpu.make_async_copy(x_hbm.at[src_v.at[r]], rows_v.at[r],
                                  sem_rows[r]).wait()

        @pl.when(c == 1)
        def _():
            pltpu.make_async_copy(xp_hbm.at[src_v.at[r]], rows_v.at[r],
                                  sem_rows[r]).wait()

    def wait_scat(r):
        pltpu.make_async_copy(rows_v.at[r], acc_sh.at[dst_v.at[r]],
                              sem_scat[r]).wait()

    issue_idx(0, 0)
    issue_idx(1, 1)
    wait_idx(0, 0)
    issue_gather(0)

    def outer(o, carry):
        for i in range(_RING):
            b = o * _RING + i
            r = i
            rn = (i + 1) % _RING
            rp = (i + 2) % _RING

            @pl.when(b >= 2)
            def _():
                wait_scat((i + 3) % _RING)

            @pl.when(b + 1 < _NB)
            def _():
                wait_idx(b + 1, rn)
                issue_gather(rn)

            @pl.when(b + 2 < _NB)
            def _():
                issue_idx(b + 2, rp)

            wait_gather(r)

            def grp(g, carry2):
                e0 = jnp.full((16,), g * 16, dtype=jnp.int32)
                for j in range(16):
                    e = g * 16 + j
                    wj = plsc.load_gather(w_v.at[r], [e0 + j])
                    for d2 in range(_D // 16):
                        dsl = pl.ds(d2 * 16, 16)
                        rows_v[r, e, dsl] = rows_v[r, e, dsl] * wj
                return carry2
            lax.fori_loop(0, _GB, grp, 0, unroll=False)

            pltpu.async_copy(rows_v.at[r], acc_sh.at[dst_v.at[r]],
                             sem_scat[r], add=True)
        return carry
    lax.fori_loop(0, _NB // _RING, outer, 0, unroll=False)

    wait_scat((_NB - 2) % _RING)
    wait_scat((_NB - 1) % _RING)

    plsc.subcore_barrier()
    pltpu.sync_copy(acc_sh.at[pl.ds(row0, _WPT)],
                    out_hbm.at[c].at[pl.ds(row0, _WPT)])

    @pl.when(s == _NS - 1)
    def _():
        pltpu.sync_copy(acc_sh.at[pl.ds(last, _XTRA)],
                        out_hbm.at[c].at[pl.ds(last, _XTRA)])


def _sc_dual_segment_sum(x, perm, src, dst, w):
    mesh = plsc.VectorSubcoreMesh(core_axis_name="c", subcore_axis_name="s")
    fn = pl.kernel(
        _sc_body,
        out_type=(
            jax.ShapeDtypeStruct((2, _N, _D), jnp.float32),
            jax.ShapeDtypeStruct((_N, _D), jnp.float32),
        ),
        mesh=mesh,
        compiler_params=pltpu.CompilerParams(needs_layout_passes=False),
        scratch_types=[
            pltpu.VMEM_SHARED((_N, _D), jnp.float32),
            pltpu.VMEM((_RING, _KB), jnp.int32),
            pltpu.VMEM((_RING, _KB), jnp.int32),
            pltpu.VMEM((_RING, _KB), jnp.float32),
            pltpu.VMEM((_RING, _KB, _D), jnp.float32),
            pltpu.VMEM((_XC, _D), jnp.float32),
            pltpu.VMEM((_XTRA, _D), jnp.float32),
            pltpu.VMEM((_WPT,), jnp.int32),
            pltpu.VMEM((_XTRA,), jnp.int32),
        ] + [pltpu.SemaphoreType.DMA] * (3 * _RING + 1),
    )
    return fn(x, perm, src, dst, w)


_BN = 1000


def _k1_body(a0_ref, a1_ref, w_ref, rid_ref, pos_ref, sp_ref, sn_ref, cnt_ref):
    i = pl.program_id(0)
    W = w_ref[...]
    ep = jnp.maximum(jnp.dot(a0_ref[...], W, preferred_element_type=jnp.float32), 0.0)
    en = jnp.maximum(jnp.dot(a1_ref[...], W, preferred_element_type=jnp.float32), 0.0)
    pos_ref[...] = ep
    rid = rid_ref[0, 0, :]
    rows = lax.broadcasted_iota(jnp.int32, (_RP, _BN), 0)
    oT = (rows == rid[None, :]).astype(jnp.float32)
    cp = jnp.dot(oT, ep, preferred_element_type=jnp.float32)
    cn = jnp.dot(oT, en, preferred_element_type=jnp.float32)
    c1 = jnp.sum(oT, axis=1)

    @pl.when(i == 0)
    def _():
        sp_ref[...] = jnp.zeros_like(sp_ref)
        sn_ref[...] = jnp.zeros_like(sn_ref)
        cnt_ref[...] = jnp.zeros_like(cnt_ref)

    sp_ref[...] += cp
    sn_ref[...] += cn
    cnt_ref[...] += c1


def _tc_encode(acc0, acc1, W_enc, rid):
    return pl.pallas_call(
        _k1_body,
        grid=(_N // _BN,),
        in_specs=[
            pl.BlockSpec((_BN, _D), lambda i: (i, 0)),
            pl.BlockSpec((_BN, _D), lambda i: (i, 0)),
            pl.BlockSpec((_D, _D), lambda i: (0, 0)),
            pl.BlockSpec((1, 1, _BN), lambda i: (i, 0, 0)),
        ],
        out_specs=[
            pl.BlockSpec((_BN, _D), lambda i: (i, 0)),
            pl.BlockSpec((_RP, _D), lambda i: (0, 0)),
            pl.BlockSpec((_RP, _D), lambda i: (0, 0)),
            pl.BlockSpec((_RP,), lambda i: (0,)),
        ],
        out_shape=[
            jax.ShapeDtypeStruct((_N, _D), jnp.float32),
            jax.ShapeDtypeStruct((_RP, _D), jnp.float32),
            jax.ShapeDtypeStruct((_RP, _D), jnp.float32),
            jax.ShapeDtypeStruct((_RP,), jnp.float32),
        ],
    )(acc0, acc1, W_enc, rid)


_EC = 1000


def _k2_body(sp_ref, sn_ref, cnt_ref, rs_ref, rd_ref, w_ref, area_ref, sim_ref,
             re_ref, ne_ref, ng_ref, city_ref):
    inv = 1.0 / jnp.maximum(cnt_ref[...], 1.0)
    mp = sp_ref[...] * inv[:, None]
    mn = sn_ref[...] * inv[:, None]
    rs = rs_ref[...]
    rd = rd_ref[...]
    cols = lax.broadcasted_iota(jnp.int32, (_EC, _RP), 1)
    rows = lax.broadcasted_iota(jnp.int32, (_RP, _EC), 0)
    raggp = jnp.zeros((_RP, _D), jnp.float32)
    raggn = jnp.zeros((_RP, _D), jnp.float32)
    for ch in range(_ER // _EC):
        rs_c = rs[ch * _EC:(ch + 1) * _EC]
        rd_c = rd[ch * _EC:(ch + 1) * _EC]
        oS = (cols == rs_c[:, None]).astype(jnp.float32)
        oDT = (rows == rd_c[None, :]).astype(jnp.float32)
        gp = jnp.dot(oS, mp, preferred_element_type=jnp.float32)
        gn = jnp.dot(oS, mn, preferred_element_type=jnp.float32)
        raggp = raggp + jnp.dot(oDT, gp, preferred_element_type=jnp.float32)
        raggn = raggn + jnp.dot(oDT, gn, preferred_element_type=jnp.float32)
    W = w_ref[...]
    re = jnp.maximum(jnp.dot(mp + raggp, W, preferred_element_type=jnp.float32), 0.0)
    ne = jnp.maximum(jnp.dot(mn + raggn, W, preferred_element_type=jnp.float32), 0.0)
    re_ref[...] = re
    ne_ref[...] = ne

    area = area_ref[...]
    wgt = area / jnp.sum(area)
    city_ref[...] = jnp.tanh(jnp.sum(re * wgt[:, None], axis=0, keepdims=True))

    sim = sim_ref[...]
    ii = lax.broadcasted_iota(jnp.int32, (_RP, _RP), 1)
    jj = lax.broadcasted_iota(jnp.int32, (_RP, _RP), 0)
    mask = (sim > 0.6) & (sim < 0.8) & (ii != jj)
    masked = jnp.where(mask, sim, -jnp.inf)
    mx = jnp.max(masked, axis=1, keepdims=True)
    has = jnp.sum(mask.astype(jnp.float32), axis=1) > 0.0
    cand = jnp.min(jnp.where(masked == mx, ii, _RP), axis=1)
    fallback = lax.rem(lax.broadcasted_iota(jnp.int32, (_RP,), 0) + 1, _R)
    neg_idx = jnp.where(has, cand, fallback)
    oN = (ii == neg_idx[:, None]).astype(jnp.float32)
    ng_ref[...] = jnp.dot(oN, re, preferred_element_type=jnp.float32)


def _tc_region(sp, sn, cnt, rsrc, rdst, W_reg, area_p, sim_p):
    return pl.pallas_call(
        _k2_body,
        out_shape=[
            jax.ShapeDtypeStruct((_RP, _D), jnp.float32),
            jax.ShapeDtypeStruct((_RP, _D), jnp.float32),
            jax.ShapeDtypeStruct((_RP, _D), jnp.float32),
            jax.ShapeDtypeStruct((1, _D), jnp.float32),
        ],
    )(sp, sn, cnt, rsrc, rdst, W_reg, area_p, sim_p)


def _k3_body(rid_ref, re_ref, ng_ref, o2_ref, o3_ref):
    rid = rid_ref[0, 0, :]
    cols = lax.broadcasted_iota(jnp.int32, (_BN, _RP), 1)
    o = (cols == rid[:, None]).astype(jnp.float32)
    o2_ref[...] = jnp.dot(o, re_ref[...], preferred_element_type=jnp.float32)
    o3_ref[...] = jnp.dot(o, ng_ref[...], preferred_element_type=jnp.float32)


def _tc_gather_out(rid, re, ng):
    return pl.pallas_call(
        _k3_body,
        grid=(_N // _BN,),
        in_specs=[
            pl.BlockSpec((1, 1, _BN), lambda i: (i, 0, 0)),
            pl.BlockSpec((_RP, _D), lambda i: (0, 0)),
            pl.BlockSpec((_RP, _D), lambda i: (0, 0)),
        ],
        out_specs=[
            pl.BlockSpec((_BN, _D), lambda i: (i, 0)),
            pl.BlockSpec((_BN, _D), lambda i: (i, 0)),
        ],
        out_shape=[
            jax.ShapeDtypeStruct((_N, _D), jnp.float32),
            jax.ShapeDtypeStruct((_N, _D), jnp.float32),
        ],
    )(rid, re, ng)


def kernel(x, edge_index, edge_weight, region_id, region_adjacency, region_area,
           coarse_region_similarity, W_enc, W_reg, weight_poi2region,
           weight_region2city):
    perm = jax.random.permutation(jax.random.key(42), _N).astype(jnp.int32)
    acc, _ = _sc_dual_segment_sum(
        x, perm, edge_index[0], edge_index[1], edge_weight)
    rid3 = region_id.reshape(_N // _BN, 1, _BN)
    pos_poi, sp, sn, cnt = _tc_encode(acc[0], acc[1], W_enc, rid3)

    area_p = jnp.pad(region_area, (0, _RP - _R))
    sim_p = jnp.pad(coarse_region_similarity, ((0, _RP - _R), (0, _RP - _R)))
    re, ne, ng, city = _tc_region(
        sp, sn, cnt, region_adjacency[0], region_adjacency[1], W_reg,
        area_p, sim_p)
    out2, out3 = _tc_gather_out(rid3, re, ng)
    return (pos_poi, out2, out3, re[:_R], ne[:_R], city[0])
